# baseline (speedup 1.0000x reference)
import functools

import jax
import jax.numpy as jnp
from jax import lax
from jax.experimental import pallas as pl
from jax.experimental.pallas import tpu as pltpu
from jax.experimental.pallas import tpu_sc as plsc

D = 256
NVALS = 4 * 8192
NW = 32
VPW = NVALS // NW
OUTW = 4 * D
WWORDS = 256 * D


@functools.partial(
    pl.kernel,
    out_type=jax.ShapeDtypeStruct((NVALS * OUTW,), jnp.float32),
    mesh=plsc.VectorSubcoreMesh(core_axis_name="c", subcore_axis_name="s"),
    scratch_types=[
        pltpu.VMEM((VPW + 16,), jnp.int32),
        pltpu.VMEM((WWORDS,), jnp.float32),
        pltpu.VMEM_SHARED((WWORDS,), jnp.float32),
        pltpu.SemaphoreType.DMA,
        pltpu.SemaphoreType.DMA,
        pltpu.SemaphoreType.DMA,
    ],
)
def _emb_kernel(xi_hbm, w_hbm, out_hbm, xi_v, w_v, sh_w, wsem, wsa, wsb):
    sid = lax.axis_index("s")
    wid = sid * 2 + lax.axis_index("c")
    vbase = wid * VPW

    WSL = WWORDS // 16
    pltpu.async_copy(
        w_hbm.at[pl.ds(sid * WSL, WSL)],
        sh_w.at[pl.ds(sid * WSL, WSL)],
        wsa)
    pltpu.sync_copy(xi_hbm.at[pl.ds(vbase, VPW)], xi_v.at[pl.ds(0, VPW)])
    pltpu.make_async_copy(
        w_hbm.at[pl.ds(0, WSL)], sh_w.at[pl.ds(0, WSL)], wsa).wait()
    plsc.subcore_barrier()

    HALF = WWORDS // 2
    pltpu.async_copy(
        sh_w.at[pl.ds(0, HALF)], w_v.at[pl.ds(0, HALF)], wsa)
    pltpu.async_copy(
        w_hbm.at[pl.ds(HALF, HALF)], w_v.at[pl.ds(HALF, HALF)], wsb)
    pltpu.make_async_copy(
        sh_w.at[pl.ds(0, HALF)], w_v.at[pl.ds(0, HALF)], wsa).wait()
    pltpu.make_async_copy(
        w_hbm.at[pl.ds(HALF, HALF)], w_v.at[pl.ds(HALF, HALF)], wsb).wait()

    def val_body(u, carry):
        w = xi_v[pl.ds(u, 16)][0]
        obase = (vbase + u) * OUTW
        for j in range(4):
            b = lax.shift_right_logical(w, jnp.int32(8 * j)) & 0xFF
            pltpu.async_copy(
                w_v.at[pl.ds(b * D, D)],
                out_hbm.at[pl.ds(obase + j * D, D)],
                wsem)
        return carry

    lax.fori_loop(0, VPW, val_body, 0)

    for _ in range(VPW * OUTW // WWORDS):
        pltpu.make_async_copy(
            w_v.at[pl.ds(0, WWORDS)],
            out_hbm.at[pl.ds(0, WWORDS)],
            wsem).wait()


def kernel(x, W):
    xi = lax.bitcast_convert_type(x, jnp.int32).reshape(-1)
    out = _emb_kernel(xi, W.reshape(-1))
    return out.reshape(x.shape[0], x.shape[1], 4 * D)


# speedup vs baseline: 1.0256x; 1.0256x over previous
import functools

import jax
import jax.numpy as jnp
from jax import lax
from jax.experimental import pallas as pl
from jax.experimental.pallas import tpu as pltpu
from jax.experimental.pallas import tpu_sc as plsc

D = 256
NVALS = 4 * 8192
NW = 32
VPW = NVALS // NW
OUTW = 4 * D
WWORDS = 256 * D


@functools.partial(
    pl.kernel,
    out_type=jax.ShapeDtypeStruct((NVALS * OUTW,), jnp.float32),
    mesh=plsc.VectorSubcoreMesh(core_axis_name="c", subcore_axis_name="s"),
    scratch_types=[
        pltpu.VMEM((VPW + 16,), jnp.int32),
        pltpu.VMEM((WWORDS,), jnp.float32),
        pltpu.VMEM_SHARED((WWORDS,), jnp.float32),
        pltpu.SemaphoreType.DMA,
        pltpu.SemaphoreType.DMA,
        pltpu.SemaphoreType.DMA,
    ],
)
def _emb_kernel(xi_hbm, w_hbm, out_hbm, xi_v, w_v, sh_w, wsem, wsa, wsb):
    sid = lax.axis_index("s")
    wid = sid * 2 + lax.axis_index("c")
    vbase = wid * VPW

    WSL = WWORDS // 16
    pltpu.async_copy(
        w_hbm.at[pl.ds(sid * WSL, WSL)],
        sh_w.at[pl.ds(sid * WSL, WSL)],
        wsa)
    pltpu.sync_copy(xi_hbm.at[pl.ds(vbase, VPW)], xi_v.at[pl.ds(0, VPW)])
    pltpu.make_async_copy(
        w_hbm.at[pl.ds(0, WSL)], sh_w.at[pl.ds(0, WSL)], wsa).wait()
    plsc.subcore_barrier()
    pltpu.sync_copy(sh_w, w_v)

    def val_body(u, carry):
        w = xi_v[pl.ds(u, 16)][0]
        obase = (vbase + u) * OUTW
        for j in range(4):
            b = lax.shift_right_logical(w, jnp.int32(8 * j)) & 0xFF
            pltpu.async_copy(
                w_v.at[pl.ds(b * D, D)],
                out_hbm.at[pl.ds(obase + j * D, D)],
                wsem)
        return carry

    lax.fori_loop(0, VPW, val_body, 0)

    for _ in range(VPW * OUTW // WWORDS):
        pltpu.make_async_copy(
            w_v.at[pl.ds(0, WWORDS)],
            out_hbm.at[pl.ds(0, WWORDS)],
            wsem).wait()


def kernel(x, W):
    xi = lax.bitcast_convert_type(x, jnp.int32).reshape(-1)
    out = _emb_kernel(xi, W.reshape(-1))
    return out.reshape(x.shape[0], x.shape[1], 4 * D)
